# baseline (speedup 1.0000x reference)
import jax
import jax.numpy as jnp
from jax import lax
from jax.experimental import pallas as pl

NB, NT, NF, NMIC = 8, 100, 128, 2
NELE = NAZI = 73
NG = NELE * NAZI
D = NF * NMIC
BT = NB * NT
SCALE = (NMIC * NF) / 2.0
CHUNK = 160
NC = BT // CHUNK


def _argmax_rows(m):
    tmax = jnp.max(m, axis=1, keepdims=True)
    iota = lax.broadcasted_iota(jnp.int32, m.shape, 1)
    idx = jnp.min(jnp.where(m == tmax, iota, NG), axis=1, keepdims=True)
    return tmax, idx


def _gather_rows(idx, t):
    iota = lax.broadcasted_iota(jnp.int32, (CHUNK, NG), 1)
    oh = jnp.where(iota == idx, 1.0, 0.0)
    return lax.dot_general(
        oh, t, (((1,), (0,)), ((), ())),
        preferred_element_type=jnp.float32,
    )


def _body(ipd_ref, t_ref, doa_ref, ss_ref, doa4_ref, vad_ref):
    ip = ipd_ref[...]
    t = t_ref[...]

    m0 = lax.dot_general(
        ip, t, (((1,), (1,)), ((), ())),
        preferred_element_type=jnp.float32,
    ) * (1.0 / SCALE)
    ss_ref[...] = m0
    _, idx0 = _argmax_rows(m0)

    tm0 = _gather_rows(idx0, t)
    num0 = jnp.sum(tm0 * ip, axis=1, keepdims=True)
    den0 = jnp.sum(tm0 * tm0, axis=1, keepdims=True)
    r0 = num0 / den0
    cur1 = ip - r0 * tm0

    m1 = lax.dot_general(
        cur1, t, (((1,), (1,)), ((), ())),
        preferred_element_type=jnp.float32,
    )
    _, idx1 = _argmax_rows(m1)

    tm1 = _gather_rows(idx1, t)
    num1 = jnp.sum(tm1 * cur1, axis=1, keepdims=True)
    den1 = jnp.sum(tm1 * tm1, axis=1, keepdims=True)
    r1 = num1 / den1

    vad_ref[...] = jnp.concatenate([r0, r1], axis=1)

    col = lax.broadcasted_iota(jnp.int32, (CHUNK, NAZI), 1)
    ele = doa_ref[0:1, :]
    azi = doa_ref[1:2, :]
    e0 = jnp.sum(jnp.where(col == idx0 // NAZI, ele, 0.0), axis=1,
                 keepdims=True)
    e1 = jnp.sum(jnp.where(col == idx1 // NAZI, ele, 0.0), axis=1,
                 keepdims=True)
    a0 = jnp.sum(jnp.where(col == idx0 % NAZI, azi, 0.0), axis=1,
                 keepdims=True)
    a1 = jnp.sum(jnp.where(col == idx1 % NAZI, azi, 0.0), axis=1,
                 keepdims=True)
    doa4_ref[...] = jnp.concatenate([e0, e1, a0, a1], axis=1)


def _pipeline(ipd, T, doa_candidate):
    return pl.pallas_call(
        _body,
        grid=(NC,),
        in_specs=[
            pl.BlockSpec((CHUNK, D), lambda i: (i, 0)),
            pl.BlockSpec((NG, D), lambda i: (0, 0)),
            pl.BlockSpec((2, NAZI), lambda i: (0, 0)),
        ],
        out_specs=[
            pl.BlockSpec((CHUNK, NG), lambda i: (i, 0)),
            pl.BlockSpec((CHUNK, 4), lambda i: (i, 0)),
            pl.BlockSpec((CHUNK, 2), lambda i: (i, 0)),
        ],
        out_shape=[
            jax.ShapeDtypeStruct((BT, NG), jnp.float32),
            jax.ShapeDtypeStruct((BT, 4), jnp.float32),
            jax.ShapeDtypeStruct((BT, 2), jnp.float32),
        ],
    )(ipd, T, doa_candidate)


def kernel(pred_ipd, dpipd_template, doa_candidate):
    pred_ipd = lax.stop_gradient(pred_ipd)
    ipd = pred_ipd.reshape(BT, D)
    T = dpipd_template.reshape(NG, D)
    ss, doa4, vad2 = _pipeline(ipd, T, doa_candidate)
    pred_ss = ss.reshape(NB, NT, NELE, NAZI)
    pred_DOAs = doa4.reshape(NB, NT, 2, 2)
    pred_VADs = vad2.reshape(NB, NT, 2)
    return (pred_DOAs, pred_VADs, pred_ss)
